# initial kernel scaffold (unmeasured)
import jax
import jax.numpy as jnp
from jax import lax
from jax.experimental import pallas as pl
from jax.experimental.pallas import tpu as pltpu

N_DEV = 4
B = 2
SQ = 128
D_MODEL = 512
SKV = 512
SKV_SHARD = 128
H_GLOBAL = 16
H_LOCAL = 4
DH = 64
BLK = 64


def kernel(x, Wq, K_ext, V_ext, Wo):
    def body(x_ref, wq_ref, k_ref, v_ref, wo_ref, out_ref,
             kv_comm, k_all, v_all, ctx_ref, ar_comm,
             kv_send, kv_recv, ar_send, ar_recv):
        p = lax.axis_index("i")
        right = lax.rem(p + 1, N_DEV)
        left = lax.rem(p + N_DEV - 1, N_DEV)

        barrier = pltpu.get_barrier_semaphore()
        for nbr in (left, right):
            pl.semaphore_signal(barrier, inc=1, device_id=(nbr,),
                                device_id_type=pl.DeviceIdType.MESH)
        pl.semaphore_wait(barrier, 2)

        k_loc = k_ref[...].astype(jnp.bfloat16)
        v_loc = v_ref[...].astype(jnp.bfloat16)
        kv_comm[0, 0:B] = k_loc
        kv_comm[0, B:2 * B] = v_loc
        k_all[:, pl.ds(p * SKV_SHARD, SKV_SHARD)] = k_loc
        v_all[:, pl.ds(p * SKV_SHARD, SKV_SHARD)] = v_loc

        def kv_hop(h):
            return pltpu.make_async_remote_copy(
                src_ref=kv_comm.at[h],
                dst_ref=kv_comm.at[h + 1],
                send_sem=kv_send.at[h],
                recv_sem=kv_recv.at[h + 1],
                device_id=(right,),
                device_id_type=pl.DeviceIdType.MESH,
            )

        rdma0 = kv_hop(0)
        rdma0.start()

        x_bf = x_ref[...].astype(jnp.bfloat16)
        wq_bf = wq_ref[...].astype(jnp.bfloat16)
        q = []
        for b in range(B):
            qb = lax.dot_general(
                x_bf[b], wq_bf, (((1,), (0,)), ((), ())),
                preferred_element_type=jnp.float32,
            ).astype(jnp.bfloat16)
            q.append(qb)

        rdma0.wait()
        for h in range(N_DEV - 1):
            if h > 0:
                r = kv_hop(h)
                r.start()
                r.wait()
            origin = lax.rem(p + N_DEV - 1 - h, N_DEV)
            k_all[:, pl.ds(origin * SKV_SHARD, SKV_SHARD)] = kv_comm[h + 1, 0:B]
            v_all[:, pl.ds(origin * SKV_SHARD, SKV_SHARD)] = kv_comm[h + 1, B:2 * B]

        row_blk = lax.broadcasted_iota(jnp.int32, (SQ, SKV), 0) // BLK
        col_blk = lax.broadcasted_iota(jnp.int32, (SQ, SKV), 1) // BLK
        mask = col_blk <= row_blk

        for b in range(B):
            for h in range(H_LOCAL):
                g = p * H_LOCAL + h
                qh = q[b][:, h * DH:(h + 1) * DH]
                kh = k_all[b, :, pl.ds(g, 1), :].reshape(SKV, DH)
                vh = v_all[b, :, pl.ds(g, 1), :].reshape(SKV, DH)
                s = lax.dot_general(
                    qh, kh, (((1,), (1,)), ((), ())),
                    preferred_element_type=jnp.float32,
                ) * 0.125
                s = jnp.where(mask, s, -1e9)
                m = jnp.max(s, axis=1, keepdims=True)
                w = jnp.exp(s - m)
                w = w / jnp.sum(w, axis=1, keepdims=True)
                ctx = lax.dot_general(
                    w.astype(jnp.bfloat16), vh, (((1,), (0,)), ((), ())),
                    preferred_element_type=jnp.float32,
                )
                ctx_ref[b, :, h * DH:(h + 1) * DH] = ctx.astype(jnp.bfloat16)

        wo_bf = wo_ref[...].astype(jnp.bfloat16)
        for b in range(B):
            pb = lax.dot_general(
                ctx_ref[b], wo_bf, (((1,), (0,)), ((), ())),
                preferred_element_type=jnp.float32,
            )
            out_ref[b] = pb
            ar_comm[0, b] = pb.astype(jnp.bfloat16)

        for h in range(N_DEV - 1):
            r = pltpu.make_async_remote_copy(
                src_ref=ar_comm.at[h],
                dst_ref=ar_comm.at[h + 1],
                send_sem=ar_send.at[h],
                recv_sem=ar_recv.at[h + 1],
                device_id=(right,),
                device_id_type=pl.DeviceIdType.MESH,
            )
            r.start()
            r.wait()
            out_ref[...] = out_ref[...] + ar_comm[h + 1].astype(jnp.float32)

    out_shape = jax.ShapeDtypeStruct((B, SQ, D_MODEL), jnp.float32)
    return pl.pallas_call(
        body,
        out_shape=out_shape,
        in_specs=[pl.BlockSpec(memory_space=pltpu.VMEM)] * 5,
        out_specs=pl.BlockSpec(memory_space=pltpu.VMEM),
        scratch_shapes=[
            pltpu.VMEM((N_DEV, 2 * B, SKV_SHARD, H_GLOBAL, DH), jnp.bfloat16),
            pltpu.VMEM((B, SKV, H_GLOBAL, DH), jnp.bfloat16),
            pltpu.VMEM((B, SKV, H_GLOBAL, DH), jnp.bfloat16),
            pltpu.VMEM((B, SQ, H_LOCAL * DH), jnp.bfloat16),
            pltpu.VMEM((N_DEV, B, SQ, D_MODEL), jnp.bfloat16),
            pltpu.SemaphoreType.DMA((N_DEV,)),
            pltpu.SemaphoreType.DMA((N_DEV,)),
            pltpu.SemaphoreType.DMA((N_DEV,)),
            pltpu.SemaphoreType.DMA((N_DEV,)),
        ],
        compiler_params=pltpu.CompilerParams(collective_id=0),
    )(x, Wq, K_ext, V_ext, Wo)


# baseline (device time: 115844 ns/iter reference)
import jax
import jax.numpy as jnp
from jax import lax
from jax.experimental import pallas as pl
from jax.experimental.pallas import tpu as pltpu

N_DEV = 4
B = 2
SQ = 128
D_MODEL = 512
SKV = 512
SKV_SHARD = 128
H_GLOBAL = 16
H_LOCAL = 4
DH = 64
BLK = 64


def kernel(x, Wq, K_ext, V_ext, Wo):
    def body(x_ref, wq_ref, k_ref, v_ref, wo_ref, out_ref,
             kv_comm, k_all, v_all, ctx_ref, ar_comm,
             kv_send, kv_recv, ar_send, ar_recv):
        p = lax.axis_index("i")
        right = lax.rem(p + 1, N_DEV)
        left = lax.rem(p + N_DEV - 1, N_DEV)

        barrier = pltpu.get_barrier_semaphore()
        for nbr in (left, right):
            pl.semaphore_signal(barrier, inc=1, device_id=(nbr,),
                                device_id_type=pl.DeviceIdType.MESH)
        pl.semaphore_wait(barrier, 2)

        k_loc = k_ref[...].astype(jnp.bfloat16)
        v_loc = v_ref[...].astype(jnp.bfloat16)
        kv_comm[0, 0:B] = k_loc
        kv_comm[0, B:2 * B] = v_loc
        k_all[:, pl.ds(p * SKV_SHARD, SKV_SHARD)] = k_loc
        v_all[:, pl.ds(p * SKV_SHARD, SKV_SHARD)] = v_loc

        def kv_hop(h):
            return pltpu.make_async_remote_copy(
                src_ref=kv_comm.at[h],
                dst_ref=kv_comm.at[h + 1],
                send_sem=kv_send.at[h],
                recv_sem=kv_recv.at[h + 1],
                device_id=(right,),
                device_id_type=pl.DeviceIdType.MESH,
            )

        rdma0 = kv_hop(0)
        rdma0.start()

        x_bf = x_ref[...].astype(jnp.bfloat16)
        wq_bf = wq_ref[...].astype(jnp.bfloat16)
        q = []
        for b in range(B):
            qb = lax.dot_general(
                x_bf[b], wq_bf, (((1,), (0,)), ((), ())),
                preferred_element_type=jnp.float32,
            ).astype(jnp.bfloat16)
            q.append(qb)

        rdma0.wait()
        for h in range(N_DEV - 1):
            if h > 0:
                r = kv_hop(h)
                r.start()
                r.wait()
            origin = lax.rem(p + N_DEV - 1 - h, N_DEV)
            k_all[:, pl.ds(origin * SKV_SHARD, SKV_SHARD)] = kv_comm[h + 1, 0:B]
            v_all[:, pl.ds(origin * SKV_SHARD, SKV_SHARD)] = kv_comm[h + 1, B:2 * B]

        row_blk = lax.broadcasted_iota(jnp.int32, (SQ, SKV), 0) // BLK
        col_blk = lax.broadcasted_iota(jnp.int32, (SQ, SKV), 1) // BLK
        mask = col_blk <= row_blk

        for b in range(B):
            for h in range(H_LOCAL):
                g = p * H_LOCAL + h
                qh = q[b][:, h * DH:(h + 1) * DH]
                hsel = lax.broadcasted_iota(jnp.int32, (SKV, H_GLOBAL, DH), 1) == g
                zero = jnp.zeros((), jnp.bfloat16)
                kh = jnp.sum(jnp.where(hsel, k_all[b], zero), axis=1)
                vh = jnp.sum(jnp.where(hsel, v_all[b], zero), axis=1)
                s = lax.dot_general(
                    qh, kh, (((1,), (1,)), ((), ())),
                    preferred_element_type=jnp.float32,
                ) * 0.125
                s = jnp.where(mask, s, -1e9)
                m = jnp.max(s, axis=1, keepdims=True)
                w = jnp.exp(s - m)
                w = w / jnp.sum(w, axis=1, keepdims=True)
                ctx = lax.dot_general(
                    w.astype(jnp.bfloat16), vh, (((1,), (0,)), ((), ())),
                    preferred_element_type=jnp.float32,
                )
                ctx_ref[b, :, h * DH:(h + 1) * DH] = ctx.astype(jnp.bfloat16)

        wo_bf = wo_ref[...].astype(jnp.bfloat16)
        for b in range(B):
            pb = lax.dot_general(
                ctx_ref[b], wo_bf, (((1,), (0,)), ((), ())),
                preferred_element_type=jnp.float32,
            )
            out_ref[b] = pb
            ar_comm[0, b] = pb.astype(jnp.bfloat16)

        for h in range(N_DEV - 1):
            r = pltpu.make_async_remote_copy(
                src_ref=ar_comm.at[h],
                dst_ref=ar_comm.at[h + 1],
                send_sem=ar_send.at[h],
                recv_sem=ar_recv.at[h + 1],
                device_id=(right,),
                device_id_type=pl.DeviceIdType.MESH,
            )
            r.start()
            r.wait()
            out_ref[...] = out_ref[...] + ar_comm[h + 1].astype(jnp.float32)

    out_shape = jax.ShapeDtypeStruct((B, SQ, D_MODEL), jnp.float32)
    return pl.pallas_call(
        body,
        out_shape=out_shape,
        in_specs=[pl.BlockSpec(memory_space=pltpu.VMEM)] * 5,
        out_specs=pl.BlockSpec(memory_space=pltpu.VMEM),
        scratch_shapes=[
            pltpu.VMEM((N_DEV, 2 * B, SKV_SHARD, H_GLOBAL, DH), jnp.bfloat16),
            pltpu.VMEM((B, SKV, H_GLOBAL, DH), jnp.bfloat16),
            pltpu.VMEM((B, SKV, H_GLOBAL, DH), jnp.bfloat16),
            pltpu.VMEM((B, SQ, H_LOCAL * DH), jnp.bfloat16),
            pltpu.VMEM((N_DEV, B, SQ, D_MODEL), jnp.bfloat16),
            pltpu.SemaphoreType.DMA((N_DEV,)),
            pltpu.SemaphoreType.DMA((N_DEV,)),
            pltpu.SemaphoreType.DMA((N_DEV,)),
            pltpu.SemaphoreType.DMA((N_DEV,)),
        ],
        compiler_params=pltpu.CompilerParams(collective_id=0),
    )(x, Wq, K_ext, V_ext, Wo)


# device time: 29729 ns/iter; 3.8967x vs baseline; 3.8967x over previous
import jax
import jax.numpy as jnp
from jax import lax
from jax.experimental import pallas as pl
from jax.experimental.pallas import tpu as pltpu

N_DEV = 4
B = 2
SQ = 128
D_MODEL = 512
SKV_SHARD = 128
H_GLOBAL = 16
H_LOCAL = 4
DH = 64
BLK = 64


def kernel(x, Wq, K_ext, V_ext, Wo):
    Kt = K_ext.transpose(0, 2, 1, 3)
    Vt = V_ext.transpose(0, 2, 1, 3)

    def body(x_ref, wq_ref, kt_ref, vt_ref, wo_ref, out_ref,
             kv_stage, kv_recv, ctx_ref, ar_stage, ar_buf,
             kv_send, kv_recv_sem, ar_send, ar_recv):
        p = lax.axis_index("i")

        barrier = pltpu.get_barrier_semaphore()
        for j in range(1, N_DEV):
            d = lax.rem(p + j, N_DEV)
            pl.semaphore_signal(barrier, inc=1, device_id=(d,),
                                device_id_type=pl.DeviceIdType.MESH)
        pl.semaphore_wait(barrier, N_DEV - 1)

        kv_descs = []
        for d in range(1, N_DEV):
            kv_descs.append(pltpu.make_async_remote_copy(
                src_ref=kv_stage.at[:, :, H_LOCAL * d:H_LOCAL * (d + 1)],
                dst_ref=kv_recv,
                send_sem=kv_send.at[d - 1],
                recv_sem=kv_recv_sem.at[0],
                device_id=(d,),
                device_id_type=pl.DeviceIdType.MESH,
            ))

        @pl.when(p == 0)
        def _():
            kv_stage[0] = kt_ref[...].astype(jnp.bfloat16)
            kv_stage[1] = vt_ref[...].astype(jnp.bfloat16)
            for r in kv_descs:
                r.start()
            kv_recv[...] = kv_stage[:, :, 0:H_LOCAL]

        x_bf = x_ref[...].astype(jnp.bfloat16)
        wq_bf = wq_ref[...].astype(jnp.bfloat16)
        q = []
        for b in range(B):
            qb = lax.dot_general(
                x_bf[b], wq_bf, (((1,), (0,)), ((), ())),
                preferred_element_type=jnp.float32,
            ).astype(jnp.bfloat16)
            q.append(qb)

        @pl.when(p != 0)
        def _():
            kv_descs[0].wait_recv()

        row_blk = lax.broadcasted_iota(jnp.int32, (SQ, SKV_SHARD), 0) // BLK
        col_blk = lax.broadcasted_iota(jnp.int32, (SQ, SKV_SHARD), 1) // BLK
        mask = col_blk <= row_blk

        for b in range(B):
            for h in range(H_LOCAL):
                qh = q[b][:, h * DH:(h + 1) * DH]
                kh = kv_recv[0, b, h]
                vh = kv_recv[1, b, h]
                s = lax.dot_general(
                    qh, kh, (((1,), (1,)), ((), ())),
                    preferred_element_type=jnp.float32,
                ) * 0.125
                s = jnp.where(mask, s, -1e9)
                m = jnp.max(s, axis=1, keepdims=True)
                w = jnp.exp(s - m)
                w = w / jnp.sum(w, axis=1, keepdims=True)
                ctx = lax.dot_general(
                    w.astype(jnp.bfloat16), vh, (((1,), (0,)), ((), ())),
                    preferred_element_type=jnp.float32,
                )
                ctx_ref[b, :, h * DH:(h + 1) * DH] = ctx.astype(jnp.bfloat16)

        wo_bf = wo_ref[...].astype(jnp.bfloat16)
        for b in range(B):
            pb = lax.dot_general(
                ctx_ref[b], wo_bf, (((1,), (0,)), ((), ())),
                preferred_element_type=jnp.float32,
            )
            out_ref[b] = pb
            ar_stage[b] = pb.astype(jnp.bfloat16)

        ar_descs = []
        for j in range(1, N_DEV):
            d = lax.rem(p + j, N_DEV)
            ar_descs.append(pltpu.make_async_remote_copy(
                src_ref=ar_stage,
                dst_ref=ar_buf.at[j - 1],
                send_sem=ar_send.at[j - 1],
                recv_sem=ar_recv.at[j - 1],
                device_id=(d,),
                device_id_type=pl.DeviceIdType.MESH,
            ))
        for r in ar_descs:
            r.start()
        for r in ar_descs:
            r.wait_recv()
        out_ref[...] = (out_ref[...]
                        + ar_buf[0].astype(jnp.float32)
                        + ar_buf[1].astype(jnp.float32)
                        + ar_buf[2].astype(jnp.float32))
        for r in ar_descs:
            r.wait_send()

        @pl.when(p == 0)
        def _():
            for r in kv_descs:
                r.wait_send()

    out_shape = jax.ShapeDtypeStruct((B, SQ, D_MODEL), jnp.float32)
    return pl.pallas_call(
        body,
        out_shape=out_shape,
        in_specs=[pl.BlockSpec(memory_space=pltpu.VMEM)] * 5,
        out_specs=pl.BlockSpec(memory_space=pltpu.VMEM),
        scratch_shapes=[
            pltpu.VMEM((2, B, H_GLOBAL, SKV_SHARD, DH), jnp.bfloat16),
            pltpu.VMEM((2, B, H_LOCAL, SKV_SHARD, DH), jnp.bfloat16),
            pltpu.VMEM((B, SQ, H_LOCAL * DH), jnp.bfloat16),
            pltpu.VMEM((B, SQ, D_MODEL), jnp.bfloat16),
            pltpu.VMEM((N_DEV - 1, B, SQ, D_MODEL), jnp.bfloat16),
            pltpu.SemaphoreType.DMA((N_DEV - 1,)),
            pltpu.SemaphoreType.DMA((1,)),
            pltpu.SemaphoreType.DMA((N_DEV - 1,)),
            pltpu.SemaphoreType.DMA((N_DEV - 1,)),
        ],
        compiler_params=pltpu.CompilerParams(collective_id=0),
    )(x, Wq, Kt, Vt, Wo)


# device time: 28953 ns/iter; 4.0011x vs baseline; 1.0268x over previous
import jax
import jax.numpy as jnp
from jax import lax
from jax.experimental import pallas as pl
from jax.experimental.pallas import tpu as pltpu

N_DEV = 4
B = 2
SQ = 128
D_MODEL = 512
SKV_SHARD = 128
H_GLOBAL = 16
H_LOCAL = 4
DH = 64
BLK = 64


def kernel(x, Wq, K_ext, V_ext, Wo):
    Kt = K_ext.astype(jnp.bfloat16).transpose(0, 2, 1, 3)
    Vt = V_ext.astype(jnp.bfloat16).transpose(0, 2, 1, 3)

    def body(x_ref, wq_ref, kt_ref, vt_ref, wo_ref, out_ref,
             kv_recv, ctx_ref, ar_stage, ar_buf,
             kv_send, kv_recv_sem, ar_send, ar_recv):
        p = lax.axis_index("i")

        barrier = pltpu.get_barrier_semaphore()
        for j in range(1, N_DEV):
            d = lax.rem(p + j, N_DEV)
            pl.semaphore_signal(barrier, inc=1, device_id=(d,),
                                device_id_type=pl.DeviceIdType.MESH)
        pl.semaphore_wait(barrier, N_DEV - 1)

        kv_descs = []
        for d in range(1, N_DEV):
            for t, src in ((0, kt_ref), (1, vt_ref)):
                kv_descs.append(pltpu.make_async_remote_copy(
                    src_ref=src.at[:, H_LOCAL * d:H_LOCAL * (d + 1)],
                    dst_ref=kv_recv.at[t],
                    send_sem=kv_send.at[2 * (d - 1) + t],
                    recv_sem=kv_recv_sem.at[t],
                    device_id=(d,),
                    device_id_type=pl.DeviceIdType.MESH,
                ))

        @pl.when(p == 0)
        def _():
            for r in kv_descs:
                r.start()
            kv_recv[0] = kt_ref[:, 0:H_LOCAL]
            kv_recv[1] = vt_ref[:, 0:H_LOCAL]

        x_bf = x_ref[...].astype(jnp.bfloat16)
        wq_bf = wq_ref[...].astype(jnp.bfloat16)
        wo_bf = wo_ref[...].astype(jnp.bfloat16)
        q = []
        for b in range(B):
            qb = lax.dot_general(
                x_bf[b], wq_bf, (((1,), (0,)), ((), ())),
                preferred_element_type=jnp.float32,
            ).astype(jnp.bfloat16)
            q.append(qb)

        @pl.when(p != 0)
        def _():
            kv_descs[0].wait_recv()
            kv_descs[1].wait_recv()

        row_blk = lax.broadcasted_iota(jnp.int32, (SQ, SKV_SHARD), 0) // BLK
        col_blk = lax.broadcasted_iota(jnp.int32, (SQ, SKV_SHARD), 1) // BLK
        mask = col_blk <= row_blk

        ar_descs = {}
        for j in range(1, N_DEV):
            d = lax.rem(p + j, N_DEV)
            for b in range(B):
                ar_descs[(j, b)] = pltpu.make_async_remote_copy(
                    src_ref=ar_stage.at[b],
                    dst_ref=ar_buf.at[j - 1, b],
                    send_sem=ar_send.at[2 * (j - 1) + b],
                    recv_sem=ar_recv.at[2 * (j - 1) + b],
                    device_id=(d,),
                    device_id_type=pl.DeviceIdType.MESH,
                )

        pbs = []
        for b in range(B):
            for h in range(H_LOCAL):
                qh = q[b][:, h * DH:(h + 1) * DH]
                kh = kv_recv[0, b, h]
                vh = kv_recv[1, b, h]
                s = lax.dot_general(
                    qh, kh, (((1,), (1,)), ((), ())),
                    preferred_element_type=jnp.float32,
                ) * 0.125
                s = jnp.where(mask, s, -1e9)
                m = jnp.max(s, axis=1, keepdims=True)
                w = jnp.exp(s - m)
                w = w / jnp.sum(w, axis=1, keepdims=True)
                ctx = lax.dot_general(
                    w.astype(jnp.bfloat16), vh, (((1,), (0,)), ((), ())),
                    preferred_element_type=jnp.float32,
                )
                ctx_ref[b, :, h * DH:(h + 1) * DH] = ctx.astype(jnp.bfloat16)
            pb = lax.dot_general(
                ctx_ref[b], wo_bf, (((1,), (0,)), ((), ())),
                preferred_element_type=jnp.float32,
            )
            ar_stage[b] = pb.astype(jnp.bfloat16)
            for j in range(1, N_DEV):
                ar_descs[(j, b)].start()
            pbs.append(pb)

        for b in range(B):
            out_ref[b] = pbs[b]

        for b in range(B):
            for j in range(1, N_DEV):
                ar_descs[(j, b)].wait_recv()
                out_ref[b] = out_ref[b] + ar_buf[j - 1, b].astype(jnp.float32)

        for b in range(B):
            for j in range(1, N_DEV):
                ar_descs[(j, b)].wait_send()

        @pl.when(p == 0)
        def _():
            for r in kv_descs:
                r.wait_send()

    out_shape = jax.ShapeDtypeStruct((B, SQ, D_MODEL), jnp.float32)
    return pl.pallas_call(
        body,
        out_shape=out_shape,
        in_specs=[pl.BlockSpec(memory_space=pltpu.VMEM)] * 5,
        out_specs=pl.BlockSpec(memory_space=pltpu.VMEM),
        scratch_shapes=[
            pltpu.VMEM((2, B, H_LOCAL, SKV_SHARD, DH), jnp.bfloat16),
            pltpu.VMEM((B, SQ, H_LOCAL * DH), jnp.bfloat16),
            pltpu.VMEM((B, SQ, D_MODEL), jnp.bfloat16),
            pltpu.VMEM((N_DEV - 1, B, SQ, D_MODEL), jnp.bfloat16),
            pltpu.SemaphoreType.DMA((2 * (N_DEV - 1),)),
            pltpu.SemaphoreType.DMA((2,)),
            pltpu.SemaphoreType.DMA((2 * (N_DEV - 1),)),
            pltpu.SemaphoreType.DMA((2 * (N_DEV - 1),)),
        ],
        compiler_params=pltpu.CompilerParams(collective_id=0),
    )(x, Wq, Kt, Vt, Wo)
